# baseline (speedup 1.0000x reference)
import functools

import jax
import jax.numpy as jnp
from jax import lax
from jax.experimental import pallas as pl
from jax.experimental.pallas import tpu as pltpu
from jax.experimental.pallas import tpu_sc as plsc

VOCAB = 1024
D = 128
BATCH = 16384
SEQ = 200
B = BATCH * SEQ

NC = 2
NS = 16
NW = NC * NS
BPW = B // NW

CH = 128
NCHUNK = BPW // CH
NBUF = 4

_mesh = plsc.VectorSubcoreMesh(core_axis_name="c", subcore_axis_name="s")


@functools.partial(
    pl.kernel,
    mesh=_mesh,
    out_type=[
        jax.ShapeDtypeStruct((B, D), jnp.float32),
        jax.ShapeDtypeStruct((B // CH, CH), jnp.int32),
    ],
    scratch_types=[
        pltpu.VMEM((NBUF, 1, CH), jnp.int32),
        pltpu.VMEM((NBUF, CH, D), jnp.float32),
        pltpu.VMEM_SHARED((VOCAB, D), jnp.float32),
        pltpu.SemaphoreType.DMA,
        pltpu.SemaphoreType.DMA,
        pltpu.SemaphoreType.DMA,
        pltpu.SemaphoreType.DMA,
    ],
)
def _sc_gather(idx_hbm, table_hbm, out_hbm, ids_out, idx_v, rows_v, tab_sh,
               sem_i, sem_g, sem_w, sem_c):
    sid = lax.axis_index("s")
    wid = sid * NC + lax.axis_index("c")
    row0 = wid * NCHUNK

    rpt = VOCAB // NS
    pltpu.sync_copy(
        table_hbm.at[pl.ds(sid * rpt, rpt)], tab_sh.at[pl.ds(sid * rpt, rpt)]
    )
    plsc.subcore_barrier()

    pltpu.async_copy(
        idx_hbm.at[pl.ds(row0, NCHUNK)], ids_out.at[pl.ds(row0, NCHUNK)], sem_c
    )

    def idx_load(i, b):
        pltpu.async_copy(idx_hbm.at[pl.ds(row0 + i, 1)], idx_v.at[b], sem_i)

    def idx_wait(b):
        pltpu.make_async_copy(
            idx_hbm.at[pl.ds(0, 1)], idx_v.at[b], sem_i
        ).wait()

    def gather_fire(b):
        pltpu.async_copy(tab_sh.at[idx_v.at[b, 0]], rows_v.at[b], sem_g)

    def gather_wait(b):
        pltpu.make_async_copy(
            table_hbm.at[pl.ds(0, CH)], rows_v.at[b], sem_g
        ).wait()

    def write_fire(i, b):
        pltpu.async_copy(
            rows_v.at[b], out_hbm.at[pl.ds((row0 + i) * CH, CH)], sem_w
        )

    def write_drain(b):
        pltpu.make_async_copy(
            rows_v.at[b], out_hbm.at[pl.ds(0, CH)], sem_w
        ).wait()

    def step(i, b, bn, drain, fire_next, load_ahead):
        if drain:
            write_drain(bn)
        if fire_next:
            idx_wait(bn)
            gather_fire(bn)
        gather_wait(b)
        write_fire(i, b)
        if load_ahead:
            idx_load(i + NBUF, b)

    for b in range(NBUF):
        idx_load(b, b)
    idx_wait(0)
    gather_fire(0)
    for i in range(NBUF - 1):
        step(i, i % NBUF, (i + 1) % NBUF,
             drain=False, fire_next=True, load_ahead=True)

    NSTEADY = (NCHUNK - NBUF - 1 - 3 + 1) // NBUF

    def body(io, carry):
        for u in range(NBUF):
            i = io * NBUF + u + 3
            b = (u + 3) % NBUF
            step(i, b, (b + 1) % NBUF,
                 drain=True, fire_next=True, load_ahead=True)
        return carry

    lax.fori_loop(0, NSTEADY, body, 0)

    for i in range(NCHUNK - NBUF - 1, NCHUNK - 1):
        step(i, i % NBUF, (i + 1) % NBUF,
             drain=True,
             fire_next=True,
             load_ahead=(i + NBUF <= NCHUNK - 1))
    i = NCHUNK - 1
    write_drain((i + 1) % NBUF)
    gather_wait(i % NBUF)
    write_fire(i, i % NBUF)
    for u in range(NBUF - 1):
        write_drain((i + 2 + u) % NBUF)
    pltpu.make_async_copy(
        idx_hbm.at[pl.ds(row0, NCHUNK)], ids_out.at[pl.ds(row0, NCHUNK)], sem_c
    ).wait()


def kernel(input_ids, table):
    ids_flat = input_ids.reshape(-1).astype(jnp.int32)
    idx2 = ids_flat.reshape(B // CH, CH)
    out, ids_out = _sc_gather(idx2, table)
    return out.reshape(BATCH, SEQ, D), ids_out.reshape(BATCH, SEQ)


# speedup vs baseline: 1.0369x; 1.0369x over previous
import functools

import jax
import jax.numpy as jnp
from jax import lax
from jax.experimental import pallas as pl
from jax.experimental.pallas import tpu as pltpu
from jax.experimental.pallas import tpu_sc as plsc

VOCAB = 1024
D = 128
BATCH = 16384
SEQ = 200
B = BATCH * SEQ

NC = 2
NS = 16
NW = NC * NS
BPW = B // NW

CH = 128
NCHUNK = BPW // CH
NBUF = 4

_mesh = plsc.VectorSubcoreMesh(core_axis_name="c", subcore_axis_name="s")


@functools.partial(
    pl.kernel,
    mesh=_mesh,
    out_type=jax.ShapeDtypeStruct((B, D), jnp.float32),
    scratch_types=[
        pltpu.VMEM((NBUF, 1, CH), jnp.int32),
        pltpu.VMEM((NBUF, CH, D), jnp.float32),
        pltpu.VMEM_SHARED((VOCAB, D), jnp.float32),
        pltpu.SemaphoreType.DMA,
        pltpu.SemaphoreType.DMA,
        pltpu.SemaphoreType.DMA,
    ],
)
def _sc_gather(idx_hbm, table_hbm, out_hbm, idx_v, rows_v, tab_sh,
               sem_i, sem_g, sem_w):
    sid = lax.axis_index("s")
    wid = sid * NC + lax.axis_index("c")
    row0 = wid * NCHUNK

    rpt = VOCAB // NS
    pltpu.sync_copy(
        table_hbm.at[pl.ds(sid * rpt, rpt)], tab_sh.at[pl.ds(sid * rpt, rpt)]
    )
    plsc.subcore_barrier()

    def idx_load(i, b):
        pltpu.async_copy(idx_hbm.at[pl.ds(row0 + i, 1)], idx_v.at[b], sem_i)

    def idx_wait(b):
        pltpu.make_async_copy(
            idx_hbm.at[pl.ds(0, 1)], idx_v.at[b], sem_i
        ).wait()

    def gather_fire(b):
        pltpu.async_copy(tab_sh.at[idx_v.at[b, 0]], rows_v.at[b], sem_g)

    def gather_wait(b):
        pltpu.make_async_copy(
            table_hbm.at[pl.ds(0, CH)], rows_v.at[b], sem_g
        ).wait()

    def write_fire(i, b):
        pltpu.async_copy(
            rows_v.at[b], out_hbm.at[pl.ds((row0 + i) * CH, CH)], sem_w
        )

    def write_drain(b):
        pltpu.make_async_copy(
            rows_v.at[b], out_hbm.at[pl.ds(0, CH)], sem_w
        ).wait()

    def step(i, b, bn, drain, fire_next, load_ahead):
        if drain:
            write_drain(bn)
        if fire_next:
            idx_wait(bn)
            gather_fire(bn)
        gather_wait(b)
        write_fire(i, b)
        if load_ahead:
            idx_load(i + NBUF, b)

    for b in range(NBUF):
        idx_load(b, b)
    idx_wait(0)
    gather_fire(0)
    for i in range(NBUF - 1):
        step(i, i % NBUF, (i + 1) % NBUF,
             drain=False, fire_next=True, load_ahead=True)

    NSTEADY = (NCHUNK - NBUF - 1 - 3 + 1) // NBUF

    def body(io, carry):
        for u in range(NBUF):
            i = io * NBUF + u + 3
            b = (u + 3) % NBUF
            step(i, b, (b + 1) % NBUF,
                 drain=True, fire_next=True, load_ahead=True)
        return carry

    lax.fori_loop(0, NSTEADY, body, 0)

    for i in range(NCHUNK - NBUF - 1, NCHUNK - 1):
        step(i, i % NBUF, (i + 1) % NBUF,
             drain=True,
             fire_next=True,
             load_ahead=(i + NBUF <= NCHUNK - 1))
    i = NCHUNK - 1
    write_drain((i + 1) % NBUF)
    gather_wait(i % NBUF)
    write_fire(i, i % NBUF)
    for u in range(NBUF - 1):
        write_drain((i + 2 + u) % NBUF)


def _tc_ids_copy(ids):
    TB = 2048

    def body(i_ref, o_ref):
        o_ref[...] = i_ref[...]

    return pl.pallas_call(
        body,
        grid=(BATCH // TB,),
        in_specs=[pl.BlockSpec((TB, SEQ), lambda i: (i, 0))],
        out_specs=pl.BlockSpec((TB, SEQ), lambda i: (i, 0)),
        out_shape=jax.ShapeDtypeStruct((BATCH, SEQ), ids.dtype),
    )(ids)


def kernel(input_ids, table):
    ids_flat = input_ids.reshape(-1).astype(jnp.int32)
    idx2 = ids_flat.reshape(B // CH, CH)
    out = _sc_gather(idx2, table)
    return out.reshape(BATCH, SEQ, D), _tc_ids_copy(input_ids)


# speedup vs baseline: 1.0446x; 1.0074x over previous
import functools

import jax
import jax.numpy as jnp
from jax import lax
from jax.experimental import pallas as pl
from jax.experimental.pallas import tpu as pltpu
from jax.experimental.pallas import tpu_sc as plsc

VOCAB = 1024
D = 128
BATCH = 16384
SEQ = 200
B = BATCH * SEQ

NC = 2
NS = 16
NW = NC * NS
BPW = B // NW

CH = 128
NCHUNK = BPW // CH
NBUF = 5

_mesh = plsc.VectorSubcoreMesh(core_axis_name="c", subcore_axis_name="s")


@functools.partial(
    pl.kernel,
    mesh=_mesh,
    out_type=jax.ShapeDtypeStruct((B, D), jnp.float32),
    scratch_types=[
        pltpu.VMEM((NBUF, 1, CH), jnp.int32),
        pltpu.VMEM((NBUF, CH, D), jnp.float32),
        pltpu.VMEM_SHARED((VOCAB, D), jnp.float32),
        pltpu.SemaphoreType.DMA,
        pltpu.SemaphoreType.DMA,
        pltpu.SemaphoreType.DMA,
    ],
)
def _sc_gather(idx_hbm, table_hbm, out_hbm, idx_v, rows_v, tab_sh,
               sem_i, sem_g, sem_w):
    sid = lax.axis_index("s")
    wid = sid * NC + lax.axis_index("c")
    row0 = wid * NCHUNK

    rpt = VOCAB // NS
    pltpu.sync_copy(
        table_hbm.at[pl.ds(sid * rpt, rpt)], tab_sh.at[pl.ds(sid * rpt, rpt)]
    )
    plsc.subcore_barrier()

    def idx_load(i, b):
        pltpu.async_copy(idx_hbm.at[pl.ds(row0 + i, 1)], idx_v.at[b], sem_i)

    def idx_wait(b):
        pltpu.make_async_copy(
            idx_hbm.at[pl.ds(0, 1)], idx_v.at[b], sem_i
        ).wait()

    def gather_fire(b):
        pltpu.async_copy(tab_sh.at[idx_v.at[b, 0]], rows_v.at[b], sem_g)

    def gather_wait(b):
        pltpu.make_async_copy(
            table_hbm.at[pl.ds(0, CH)], rows_v.at[b], sem_g
        ).wait()

    def write_fire(i, b):
        pltpu.async_copy(
            rows_v.at[b], out_hbm.at[pl.ds((row0 + i) * CH, CH)], sem_w
        )

    def write_drain(b):
        pltpu.make_async_copy(
            rows_v.at[b], out_hbm.at[pl.ds(0, CH)], sem_w
        ).wait()

    def step(i, b, bn, drain, fire_next, load_ahead):
        if drain:
            write_drain(bn)
        if fire_next:
            idx_wait(bn)
            gather_fire(bn)
        gather_wait(b)
        write_fire(i, b)
        if load_ahead:
            idx_load(i + NBUF, b)

    for b in range(NBUF):
        idx_load(b, b)
    idx_wait(0)
    gather_fire(0)
    for i in range(NBUF - 1):
        step(i, i % NBUF, (i + 1) % NBUF,
             drain=False, fire_next=True, load_ahead=True)

    NSTEADY = (NCHUNK - 2 * NBUF) // NBUF

    def body(io, carry):
        for u in range(NBUF):
            i = io * NBUF + u + (NBUF - 1)
            b = (u + NBUF - 1) % NBUF
            step(i, b, (b + 1) % NBUF,
                 drain=True, fire_next=True, load_ahead=True)
        return carry

    lax.fori_loop(0, NSTEADY, body, 0)

    for i in range(NCHUNK - NBUF - 1, NCHUNK - 1):
        step(i, i % NBUF, (i + 1) % NBUF,
             drain=True,
             fire_next=True,
             load_ahead=(i + NBUF <= NCHUNK - 1))
    i = NCHUNK - 1
    write_drain((i + 1) % NBUF)
    gather_wait(i % NBUF)
    write_fire(i, i % NBUF)
    for u in range(NBUF - 1):
        write_drain((i + 2 + u) % NBUF)


def kernel(input_ids, table):
    ids_flat = input_ids.reshape(-1).astype(jnp.int32)
    idx2 = ids_flat.reshape(B // CH, CH)
    out = _sc_gather(idx2, table)
    return out.reshape(BATCH, SEQ, D), input_ids


# speedup vs baseline: 1.0452x; 1.0006x over previous
import functools

import jax
import jax.numpy as jnp
from jax import lax
from jax.experimental import pallas as pl
from jax.experimental.pallas import tpu as pltpu
from jax.experimental.pallas import tpu_sc as plsc

VOCAB = 1024
D = 128
BATCH = 16384
SEQ = 200
B = BATCH * SEQ

NC = 2
NS = 16
NW = NC * NS
BPW = B // NW

CH = 128
NCHUNK = BPW // CH
NBUF = 4

_mesh = plsc.VectorSubcoreMesh(core_axis_name="c", subcore_axis_name="s")


@functools.partial(
    pl.kernel,
    mesh=_mesh,
    out_type=jax.ShapeDtypeStruct((B, D), jnp.float32),
    scratch_types=[
        pltpu.VMEM((NBUF, 1, CH), jnp.int32),
        pltpu.VMEM((NBUF, CH, D), jnp.float32),
        pltpu.VMEM_SHARED((VOCAB, D), jnp.float32),
        pltpu.SemaphoreType.DMA,
        pltpu.SemaphoreType.DMA,
        pltpu.SemaphoreType.DMA,
    ],
)
def _sc_gather(idx_hbm, table_hbm, out_hbm, idx_v, rows_v, tab_sh,
               sem_i, sem_g, sem_w):
    sid = lax.axis_index("s")
    wid = sid * NC + lax.axis_index("c")
    row0 = wid * NCHUNK

    rpt = VOCAB // NS
    pltpu.sync_copy(
        table_hbm.at[pl.ds(sid * rpt, rpt)], tab_sh.at[pl.ds(sid * rpt, rpt)]
    )
    plsc.subcore_barrier()

    def idx_load(i, b):
        pltpu.async_copy(idx_hbm.at[pl.ds(row0 + i, 1)], idx_v.at[b], sem_i)

    def idx_wait(b):
        pltpu.make_async_copy(
            idx_hbm.at[pl.ds(0, 1)], idx_v.at[b], sem_i
        ).wait()

    def gather_fire(b):
        pltpu.async_copy(tab_sh.at[idx_v.at[b, 0]], rows_v.at[b], sem_g)

    def gather_wait(b):
        pltpu.make_async_copy(
            table_hbm.at[pl.ds(0, CH)], rows_v.at[b], sem_g
        ).wait()

    def write_fire(i, b):
        pltpu.async_copy(
            rows_v.at[b], out_hbm.at[pl.ds((row0 + i) * CH, CH)], sem_w
        )

    def write_drain(b):
        pltpu.make_async_copy(
            rows_v.at[b], out_hbm.at[pl.ds(0, CH)], sem_w
        ).wait()

    def step(i, b, bn, drain, fire_next, load_ahead):
        if drain:
            write_drain(bn)
        if fire_next:
            idx_wait(bn)
            gather_fire(bn)
        gather_wait(b)
        write_fire(i, b)
        if load_ahead:
            idx_load(i + NBUF, b)

    for b in range(NBUF):
        idx_load(b, b)
    idx_wait(0)
    gather_fire(0)
    for i in range(NBUF - 1):
        step(i, i % NBUF, (i + 1) % NBUF,
             drain=False, fire_next=True, load_ahead=True)

    NSTEADY = (NCHUNK - 2 * NBUF) // NBUF

    def body(io, carry):
        for u in range(NBUF):
            i = io * NBUF + u + (NBUF - 1)
            b = (u + NBUF - 1) % NBUF
            step(i, b, (b + 1) % NBUF,
                 drain=True, fire_next=True, load_ahead=True)
        return carry

    lax.fori_loop(0, NSTEADY, body, 0)

    for i in range(NCHUNK - NBUF - 1, NCHUNK - 1):
        step(i, i % NBUF, (i + 1) % NBUF,
             drain=True,
             fire_next=True,
             load_ahead=(i + NBUF <= NCHUNK - 1))
    i = NCHUNK - 1
    write_drain((i + 1) % NBUF)
    gather_wait(i % NBUF)
    write_fire(i, i % NBUF)
    for u in range(NBUF - 1):
        write_drain((i + 2 + u) % NBUF)


def kernel(input_ids, table):
    ids_flat = input_ids.reshape(-1).astype(jnp.int32)
    idx2 = ids_flat.reshape(B // CH, CH)
    out = _sc_gather(idx2, table)
    return out.reshape(BATCH, SEQ, D), input_ids
